# Initial kernel scaffold; baseline (speedup 1.0000x reference)
import jax
import jax.numpy as jnp
from jax.experimental import pallas as pl


def kernel(x, edge_index, W):
    raise NotImplementedError("write your pallas kernel here")


# speedup vs baseline: 3.0001x; 3.0001x over previous
import functools

import jax
import jax.numpy as jnp
from jax import lax
from jax.experimental import pallas as pl
from jax.experimental.pallas import tpu as pltpu
from jax.experimental.pallas import tpu_sc as plsc

_D = 4
_N = 1024
_E = 16384
_DD = _D * _D
_ROWS = _N * _N
_NC = 2
_NS = 16
_L = 16
_PAD = _NC * _NS
_HALF = _ROWS // _NC
_ZROWS = _HALF // _NS
_ZBUF = 2048
_NZDMA = _ZROWS // _ZBUF
_EPS = _E // _NS
_NCHUNK = _EPS // _L
_SCAT = 128
_NSCAT = _EPS // _SCAT


@functools.partial(
    pl.kernel,
    out_type=jax.ShapeDtypeStruct((_ROWS + _PAD, _DD), jnp.float32),
    mesh=plsc.VectorSubcoreMesh(core_axis_name="c", subcore_axis_name="s",
                                num_cores=_NC, num_subcores=_NS),
    compiler_params=pltpu.CompilerParams(needs_layout_passes=False,
                                         use_tc_tiling_on_sc=False),
    scratch_types=[
        pltpu.VMEM((_N, _D), jnp.float32),
        pltpu.VMEM((_DD * _D,), jnp.float32),
        pltpu.VMEM((_EPS,), jnp.int32),
        pltpu.VMEM((_EPS,), jnp.int32),
        pltpu.VMEM((_NSCAT, _SCAT), jnp.int32),
        pltpu.VMEM((_EPS, _DD), jnp.float32),
        pltpu.VMEM((_ZBUF, _DD), jnp.float32),
        pltpu.SemaphoreType.DMA,
        pltpu.SemaphoreType.DMA,
        pltpu.SemaphoreType.DMA,
    ],
)
def _sheaf_sc(x_hbm, src_hbm, dst_hbm, w_hbm, out_hbm,
              x_v, w_v, src_v, dst_v, flat_v, rows_v, z_v,
              in_sem, z_sem, s_sem):
    cid = lax.axis_index("c")
    sid = lax.axis_index("s")

    ebase = sid * _EPS
    in_copies = [
        pltpu.async_copy(x_hbm, x_v, in_sem),
        pltpu.async_copy(w_hbm, w_v, in_sem),
        pltpu.async_copy(src_hbm.at[pl.ds(ebase, _EPS)], src_v, in_sem),
        pltpu.async_copy(dst_hbm.at[pl.ds(ebase, _EPS)], dst_v, in_sem),
    ]

    zvec = jnp.zeros((_L,), jnp.float32)

    def _zinit(i, carry):
        z_v[i, :] = zvec
        return carry

    lax.fori_loop(0, _ZBUF, _zinit, 0)

    zbase = cid * _HALF + sid * _ZROWS
    z_copies = []
    for j in range(_NZDMA):
        z_copies.append(
            pltpu.async_copy(z_v, out_hbm.at[pl.ds(zbase + j * _ZBUF, _ZBUF)],
                             z_sem))

    for c in in_copies:
        c.wait()

    wvecs = [w_v[pl.ds(j * _L, _L)] for j in range(_DD * _D // _L)]
    wsc = [wvecs[i // _L][i % _L] for i in range(_DD * _D)]
    lanes = lax.iota(jnp.int32, _L)
    lo = cid * (_N // _NC)
    trash = _ROWS + cid * _NS + sid

    def _chunk(i, carry):
        s16 = src_v[pl.ds(i * _L, _L)]
        d16 = dst_v[pl.ds(i * _L, _L)]
        flat = s16 * _N + d16
        keep = (s16 >= lo) & (s16 < lo + _N // _NC)
        flat = jnp.where(keep, flat, trash)
        diffs = []
        for k in range(_D):
            kk = jnp.full((_L,), k, jnp.int32)
            a = plsc.load_gather(x_v, [s16, kk])
            b = plsc.load_gather(x_v, [d16, kk])
            diffs.append(jnp.abs(a - b))
        ridx = i * _L + lanes
        for c in range(_DD):
            acc = diffs[0] * wsc[c * _D]
            for k in range(1, _D):
                acc = acc + diffs[k] * wsc[c * _D + k]
            plsc.store_scatter(rows_v, [ridx, jnp.full((_L,), c, jnp.int32)],
                               acc)
        flat_v[i // (_SCAT // _L), pl.ds((i % (_SCAT // _L)) * _L, _L)] = flat
        return carry

    lax.fori_loop(0, _NCHUNK, _chunk, 0)

    for c in z_copies:
        c.wait()
    plsc.subcore_barrier()

    s_copies = []
    for j in range(_NSCAT):
        s_copies.append(
            pltpu.async_copy(rows_v.at[pl.ds(j * _SCAT, _SCAT)],
                             out_hbm.at[flat_v.at[j]], s_sem))
    for c in s_copies:
        c.wait()


def kernel(x, edge_index, W):
    ei = edge_index.astype(jnp.int32)
    out = _sheaf_sc(x, ei[0], ei[1], W.reshape(-1))
    return out[:_ROWS].reshape(_N, _N, _D, _D)


# speedup vs baseline: 4.5478x; 1.5159x over previous
import functools

import jax
import jax.numpy as jnp
from jax import lax
from jax.experimental import pallas as pl
from jax.experimental.pallas import tpu as pltpu
from jax.experimental.pallas import tpu_sc as plsc

_D = 4
_N = 1024
_E = 16384
_DD = _D * _D
_ROWS = _N * _N
_NC = 2
_NS = 16
_L = 16
_HALF = _ROWS // _NC
_ZROWS = _HALF // _NS
_ZBUF = 2048
_NZDMA = _ZROWS // _ZBUF
_EPS = _E // _NS
_NCHUNK = _EPS // _L
_SCAT = 128
_NSCAT = _EPS // _SCAT


@functools.partial(
    pl.kernel,
    out_type=jax.ShapeDtypeStruct((_ROWS, _DD), jnp.float32),
    mesh=plsc.VectorSubcoreMesh(core_axis_name="c", subcore_axis_name="s",
                                num_cores=_NC, num_subcores=_NS),
    compiler_params=pltpu.CompilerParams(needs_layout_passes=False,
                                         use_tc_tiling_on_sc=False),
    scratch_types=[
        pltpu.VMEM((_N, _D), jnp.float32),
        pltpu.VMEM((_DD * _D,), jnp.float32),
        pltpu.VMEM((_EPS,), jnp.int32),
        pltpu.VMEM((_EPS,), jnp.int32),
        pltpu.VMEM((_NSCAT, _SCAT), jnp.int32),
        pltpu.VMEM((_EPS, _DD), jnp.float32),
        pltpu.VMEM((_ZBUF, _DD), jnp.float32),
        pltpu.SemaphoreType.DMA,
        pltpu.SemaphoreType.DMA,
        pltpu.SemaphoreType.DMA,
    ],
)
def _sheaf_sc(x_hbm, src_hbm, dst_hbm, w_hbm, out_hbm,
              x_v, w_v, src_v, dst_v, flat_v, rows_v, z_v,
              in_sem, z_sem, s_sem):
    cid = lax.axis_index("c")
    sid = lax.axis_index("s")

    ebase = sid * _EPS
    in_copies = [
        pltpu.async_copy(x_hbm, x_v, in_sem),
        pltpu.async_copy(w_hbm, w_v, in_sem),
        pltpu.async_copy(src_hbm.at[pl.ds(ebase, _EPS)], src_v, in_sem),
        pltpu.async_copy(dst_hbm.at[pl.ds(ebase, _EPS)], dst_v, in_sem),
    ]

    zvec = jnp.zeros((_L,), jnp.float32)

    def _zinit(i, carry):
        z_v[i, :] = zvec
        return carry

    lax.fori_loop(0, _ZBUF, _zinit, 0)

    zbase = cid * _HALF + sid * _ZROWS
    z_copies = []
    for j in range(_NZDMA):
        z_copies.append(
            pltpu.async_copy(z_v, out_hbm.at[pl.ds(zbase + j * _ZBUF, _ZBUF)],
                             z_sem))

    for c in in_copies:
        c.wait()

    wvecs = [w_v[pl.ds(j * _L, _L)] for j in range(_DD * _D // _L)]
    wsc = [wvecs[i // _L][i % _L] for i in range(_DD * _D)]
    lanes = lax.iota(jnp.int32, _L)

    def _chunk(i, carry):
        s16 = src_v[pl.ds(i * _L, _L)]
        d16 = dst_v[pl.ds(i * _L, _L)]
        flat = s16 * _N + d16
        diffs = []
        for k in range(_D):
            kk = jnp.full((_L,), k, jnp.int32)
            a = plsc.load_gather(x_v, [s16, kk])
            b = plsc.load_gather(x_v, [d16, kk])
            diffs.append(jnp.abs(a - b))
        ridx = i * _L + lanes
        for c in range(_DD):
            acc = diffs[0] * wsc[c * _D]
            for k in range(1, _D):
                acc = acc + diffs[k] * wsc[c * _D + k]
            plsc.store_scatter(rows_v, [ridx, jnp.full((_L,), c, jnp.int32)],
                               acc)
        flat_v[i // (_SCAT // _L), pl.ds((i % (_SCAT // _L)) * _L, _L)] = flat
        return carry

    lax.fori_loop(0, _NCHUNK, _chunk, 0)

    for c in z_copies:
        c.wait()
    plsc.subcore_barrier()

    s_copies = []
    for j in range(_NSCAT):
        s_copies.append(
            pltpu.async_copy(rows_v.at[pl.ds(j * _SCAT, _SCAT)],
                             out_hbm.at[flat_v.at[j]], s_sem))
    for c in s_copies:
        c.wait()


def kernel(x, edge_index, W):
    ei = edge_index.astype(jnp.int32)
    out = _sheaf_sc(x, ei[0], ei[1], W.reshape(-1))
    return out.reshape(_N, _N, _D, _D)


# speedup vs baseline: 5.4881x; 1.2067x over previous
import functools

import jax
import jax.numpy as jnp
from jax import lax
from jax.experimental import pallas as pl
from jax.experimental.pallas import tpu as pltpu
from jax.experimental.pallas import tpu_sc as plsc

_D = 4
_N = 1024
_E = 16384
_DD = _D * _D
_ELEMS = _N * _N * _DD
_NC = 2
_NS = 16
_L = 16
_HALF = _ELEMS // _NC
_ZELEM = _HALF // _NS
_ZBUF = 32768
_NZDMA = _ZELEM // _ZBUF
_EPS = _E // _NS
_NCHUNK = _EPS // _L
_NELEM = _EPS * _DD
_IB = 128
_IK = 16
_NSCAT = _NELEM // (_IK * _IB)


@functools.partial(
    pl.kernel,
    out_type=jax.ShapeDtypeStruct((_ELEMS,), jnp.float32),
    mesh=plsc.VectorSubcoreMesh(core_axis_name="c", subcore_axis_name="s",
                                num_cores=_NC, num_subcores=_NS),
    compiler_params=pltpu.CompilerParams(needs_layout_passes=False,
                                         use_tc_tiling_on_sc=False),
    scratch_types=[
        pltpu.VMEM((_N, _D), jnp.float32),
        pltpu.VMEM((_DD * _D,), jnp.float32),
        pltpu.VMEM((_EPS,), jnp.int32),
        pltpu.VMEM((_EPS,), jnp.int32),
        pltpu.VMEM((_NELEM,), jnp.int32),
        pltpu.VMEM((_NELEM,), jnp.float32),
        pltpu.VMEM((_ZBUF,), jnp.float32),
        pltpu.SemaphoreType.DMA,
        pltpu.SemaphoreType.DMA,
        pltpu.SemaphoreType.DMA,
    ],
)
def _sheaf_sc(x_hbm, src_hbm, dst_hbm, w_hbm, out_hbm,
              x_v, w_v, src_v, dst_v, idx_v, val_v, z_v,
              in_sem, z_sem, s_sem):
    cid = lax.axis_index("c")
    sid = lax.axis_index("s")

    ebase = sid * _EPS
    in_copies = [
        pltpu.async_copy(x_hbm, x_v, in_sem),
        pltpu.async_copy(w_hbm, w_v, in_sem),
        pltpu.async_copy(src_hbm.at[pl.ds(ebase, _EPS)], src_v, in_sem),
        pltpu.async_copy(dst_hbm.at[pl.ds(ebase, _EPS)], dst_v, in_sem),
    ]

    zvec = jnp.zeros((_L,), jnp.float32)

    def _zinit(i, carry):
        z_v[pl.ds(i * _L, _L)] = zvec
        return carry

    lax.fori_loop(0, _ZBUF // _L, _zinit, 0)

    zbase = cid * _HALF + sid * _ZELEM
    z_copies = []
    for j in range(_NZDMA):
        z_copies.append(
            pltpu.async_copy(z_v, out_hbm.at[pl.ds(zbase + j * _ZBUF, _ZBUF)],
                             z_sem))

    for c in in_copies:
        c.wait()

    wvecs = [w_v[pl.ds(j * _L, _L)] for j in range(_DD * _D // _L)]
    wsc = [wvecs[i // _L][i % _L] for i in range(_DD * _D)]
    lanes = lax.iota(jnp.int32, _L)

    def _chunk(i, carry):
        s16 = src_v[pl.ds(i * _L, _L)]
        d16 = dst_v[pl.ds(i * _L, _L)]
        abase = s16 * (_DD * _N) + (d16 >> 7) * 512 + (d16 & 127)
        diffs = []
        for k in range(_D):
            kk = jnp.full((_L,), k, jnp.int32)
            a = plsc.load_gather(x_v, [s16, kk])
            b = plsc.load_gather(x_v, [d16, kk])
            diffs.append(jnp.abs(a - b))
        for c in range(_DD):
            acc = diffs[0] * wsc[c * _D]
            for k in range(1, _D):
                acc = acc + diffs[k] * wsc[c * _D + k]
            slot = pl.ds(c * _EPS + i * _L, _L)
            val_v[slot] = acc
            ci, cj = c // _D, c % _D
            idx_v[slot] = abase + (ci * _D * _N + cj * 128)
        return carry

    lax.fori_loop(0, _NCHUNK, _chunk, 0)

    for c in z_copies:
        c.wait()
    plsc.subcore_barrier()

    pltpu.async_copy(val_v, out_hbm.at[idx_v], s_sem).wait()


def kernel(x, edge_index, W):
    ei = edge_index.astype(jnp.int32)
    raw = _sheaf_sc(x, ei[0], ei[1], W.reshape(-1))
    r5 = raw.reshape(_N, _D, _N // 128, _D, 128)
    return r5.transpose(0, 2, 4, 1, 3).reshape(_N, _N, _D, _D)


# speedup vs baseline: 5.4904x; 1.0004x over previous
import functools

import jax
import jax.numpy as jnp
from jax import lax
from jax.experimental import pallas as pl
from jax.experimental.pallas import tpu as pltpu
from jax.experimental.pallas import tpu_sc as plsc

_D = 4
_N = 1024
_E = 16384
_DD = _D * _D
_ELEMS = _N * _N * _DD
_NC = 2
_NS = 16
_L = 16
_HALF = _ELEMS // _NC
_ZELEM = _HALF // _NS
_ZBUF = 32768
_NZDMA = _ZELEM // _ZBUF
_EPS = _E // _NS
_NCHUNK = _EPS // _L
_NELEM = _EPS * _DD
_IB = 128
_IK = 16
_NSCAT = _NELEM // (_IK * _IB)


@functools.partial(
    pl.kernel,
    out_type=jax.ShapeDtypeStruct((_ELEMS,), jnp.float32),
    mesh=plsc.VectorSubcoreMesh(core_axis_name="c", subcore_axis_name="s",
                                num_cores=_NC, num_subcores=_NS),
    compiler_params=pltpu.CompilerParams(needs_layout_passes=False,
                                         use_tc_tiling_on_sc=False),
    scratch_types=[
        pltpu.VMEM((_N, _D), jnp.float32),
        pltpu.VMEM((_DD * _D,), jnp.float32),
        pltpu.VMEM((_EPS,), jnp.int32),
        pltpu.VMEM((_EPS,), jnp.int32),
        pltpu.VMEM((_NELEM // _IB, _IB), jnp.int32),
        pltpu.VMEM((_NELEM // _IB, _IB), jnp.float32),
        pltpu.VMEM((_ZBUF,), jnp.float32),
        pltpu.SemaphoreType.DMA,
        pltpu.SemaphoreType.DMA,
        pltpu.SemaphoreType.DMA,
    ],
)
def _sheaf_sc(x_hbm, src_hbm, dst_hbm, w_hbm, out_hbm,
              x_v, w_v, src_v, dst_v, idx_v, val_v, z_v,
              in_sem, z_sem, s_sem):
    cid = lax.axis_index("c")
    sid = lax.axis_index("s")

    ebase = sid * _EPS
    in_copies = [
        pltpu.async_copy(x_hbm, x_v, in_sem),
        pltpu.async_copy(w_hbm, w_v, in_sem),
        pltpu.async_copy(src_hbm.at[pl.ds(ebase, _EPS)], src_v, in_sem),
        pltpu.async_copy(dst_hbm.at[pl.ds(ebase, _EPS)], dst_v, in_sem),
    ]

    zvec = jnp.zeros((_L,), jnp.float32)

    def _zinit(i, carry):
        z_v[pl.ds(i * _L, _L)] = zvec
        return carry

    lax.fori_loop(0, _ZBUF // _L, _zinit, 0)

    zbase = cid * _HALF + sid * _ZELEM
    z_copies = []
    for j in range(_NZDMA):
        z_copies.append(
            pltpu.async_copy(z_v, out_hbm.at[pl.ds(zbase + j * _ZBUF, _ZBUF)],
                             z_sem))

    for c in in_copies:
        c.wait()

    wvecs = [w_v[pl.ds(j * _L, _L)] for j in range(_DD * _D // _L)]
    wsc = [wvecs[i // _L][i % _L] for i in range(_DD * _D)]
    lanes = lax.iota(jnp.int32, _L)

    def _chunk(i, carry):
        s16 = src_v[pl.ds(i * _L, _L)]
        d16 = dst_v[pl.ds(i * _L, _L)]
        abase = s16 * (_DD * _N) + (d16 >> 7) * 512 + (d16 & 127)
        diffs = []
        for k in range(_D):
            kk = jnp.full((_L,), k, jnp.int32)
            a = plsc.load_gather(x_v, [s16, kk])
            b = plsc.load_gather(x_v, [d16, kk])
            diffs.append(jnp.abs(a - b))
        for c in range(_DD):
            acc = diffs[0] * wsc[c * _D]
            for k in range(1, _D):
                acc = acc + diffs[k] * wsc[c * _D + k]
            row = c * (_EPS // _IB) + i // 8
            col = pl.ds((i % 8) * _L, _L)
            val_v[row, col] = acc
            ci, cj = c // _D, c % _D
            idx_v[row, col] = abase + (ci * _D * _N + cj * 128)
        return carry

    lax.fori_loop(0, _NCHUNK, _chunk, 0)

    for c in z_copies:
        c.wait()
    plsc.subcore_barrier()

    def _scat(j, carry):
        pltpu.async_copy(val_v.at[j], out_hbm.at[idx_v.at[j]], s_sem)
        return carry

    lax.fori_loop(0, _NELEM // _IB, _scat, 0)

    def _drain(j, carry):
        pltpu.make_async_copy(val_v.at[j], out_hbm.at[idx_v.at[j]],
                              s_sem).wait()
        return carry

    lax.fori_loop(0, _NELEM // _IB, _drain, 0)


def kernel(x, edge_index, W):
    ei = edge_index.astype(jnp.int32)
    raw = _sheaf_sc(x, ei[0], ei[1], W.reshape(-1))
    r5 = raw.reshape(_N, _D, _N // 128, _D, 128)
    return r5.transpose(0, 2, 4, 1, 3).reshape(_N, _N, _D, _D)
